# baseline (speedup 1.0000x reference)
import functools

import jax
import jax.numpy as jnp
from jax import lax
from jax.experimental import pallas as pl
from jax.experimental.pallas import tpu as pltpu
from jax.experimental.pallas import tpu_sc as plsc

_N = 100000
_CH = 128
_NT = 16
_NW = 32
_R = 3200
_N2 = _NW * _R
_CHUNK = 128
_NCH = _R // _CHUNK
_BE = 20000
_GE = _N // _BE
_BM = 640
_GM = _R // _BM

_DN = (((0,), (0,)), ((), ()))

_mesh = lambda: plsc.VectorSubcoreMesh(core_axis_name="c", subcore_axis_name="s")


def _sc_compact_gather(nt_hbm, x_hbm, itemv_hbm, idx_out, cnt_out, xc_out,
                       ntv, itv, idx_loc, xbuf, cntv, sem):
    wid = lax.axis_index("s") * 2 + lax.axis_index("c")
    base = wid * _R
    pltpu.sync_copy(itemv_hbm, itv)
    item_vec = itv[...]
    pltpu.sync_copy(nt_hbm.at[pl.ds(base, _R)], ntv)

    def chunk_body(c, cnt_vec):
        for j in range(_CHUNK // 16):
            v = ntv[pl.ds(c * _CHUNK + j * 16, 16)]
            m = v == item_vec
            pos = cnt_vec + plsc.cumsum(m.astype(jnp.int32)) - 1
            glob = base + c * _CHUNK + j * 16 + lax.iota(jnp.int32, 16)
            plsc.store_scatter(idx_loc, [pos >> 7, pos & 127], glob, mask=m)
            cnt_vec = cnt_vec + plsc.all_reduce_population_count(m)
        return cnt_vec
    cnt_vec = lax.fori_loop(0, _NCH, chunk_body, jnp.zeros((16,), jnp.int32))
    cnt = jnp.max(cnt_vec)

    @pl.when(cnt > 0)
    def _pad():
        lvec = jnp.full((16,), cnt - 1, jnp.int32)
        last = plsc.load_gather(idx_loc, [lvec >> 7, lvec & 127])
        astart = (cnt // 16) * 16
        pad_end = ((cnt + _CHUNK - 1) // _CHUNK) * _CHUNK
        for k in range(9):
            pos = astart + k * 16 + lax.iota(jnp.int32, 16)
            m = (pos >= cnt) & (pos < pad_end)
            plsc.store_scatter(idx_loc, [pos >> 7, pos & 127], last, mask=m)

    cntv[...] = cnt_vec
    pltpu.sync_copy(cntv, cnt_out.at[pl.ds(wid * 16, 16)])
    pltpu.sync_copy(idx_loc.at[pl.ds(0, _NCH)], idx_out.at[wid])

    def gather_body(c, _):
        @pl.when(c * _CHUNK < cnt)
        def _go():
            pltpu.async_copy(x_hbm.at[idx_loc.at[c]], xbuf, sem).wait()
            pltpu.sync_copy(xbuf, xc_out.at[pl.ds(base + c * _CHUNK, _CHUNK)])
        return 0
    lax.fori_loop(0, _NCH, gather_body, 0)


def _sc_scatter(out_ref, linc_hbm, idx_hbm, cnt_hbm, idx2, rows, cntv, sem):
    wid = lax.axis_index("s") * 2 + lax.axis_index("c")
    base = wid * _R
    pltpu.sync_copy(cnt_hbm.at[pl.ds(wid * 16, 16)], cntv)
    cnt = jnp.max(cntv[...])
    pltpu.sync_copy(idx_hbm.at[wid], idx2)

    def scatter_body(c, _):
        @pl.when(c * _CHUNK < cnt)
        def _go():
            pltpu.sync_copy(linc_hbm.at[pl.ds(base + c * _CHUNK, _CHUNK)], rows)
            pltpu.async_copy(rows, out_ref.at[idx2.at[c]], sem).wait()
        return 0
    lax.fori_loop(0, _NCH, scatter_body, 0)


def _tc_emb_body(nt_ref, emb_ref, out_ref):
    nt_row = nt_ref[0]
    ohT = (nt_row == lax.broadcasted_iota(jnp.int32, (_NT, _BE), 0)
           ).astype(jnp.float32)
    out_ref[...] = lax.dot_general(ohT, emb_ref[...], _DN,
                                   preferred_element_type=jnp.float32)


def _tc_mm_body(cnt_ref, x_ref, wt_ref, b_ref, o_ref):
    w = pl.program_id(0)
    j = pl.program_id(1)
    jmax = jnp.maximum((cnt_ref[w * 16] + _BM - 1) // _BM - 1, 0)

    @pl.when(j <= jmax)
    def _go():
        o_ref[...] = (jnp.dot(x_ref[...], wt_ref[...],
                              preferred_element_type=jnp.float32) + b_ref[0, :])


def _mm_index(w, j, cnt_ref):
    jmax = jnp.maximum((cnt_ref[w * 16] + _BM - 1) // _BM - 1, 0)
    return (w * _GM + jnp.minimum(j, jmax), 0)


def kernel(x, node_type, item_id, emb_weight, W, b):
    item32 = jnp.asarray(item_id, jnp.int32)
    itemv = jnp.full((16,), item32, jnp.int32)
    padv = (item32 + 1) & (_NT - 1)
    nt_pad = jnp.concatenate(
        [node_type, jnp.full((_N2 - _N,), padv, jnp.int32)])
    wt = W.T
    b2 = b.reshape(1, _CH)

    sc_a = pl.kernel(
        _sc_compact_gather,
        out_type=(
            jax.ShapeDtypeStruct((_NW, _NCH, _CHUNK), jnp.int32),
            jax.ShapeDtypeStruct((_NW * 16,), jnp.int32),
            jax.ShapeDtypeStruct((_N2, _CH), jnp.float32),
        ),
        mesh=_mesh(),
        scratch_types=[
            pltpu.VMEM((_R,), jnp.int32),
            pltpu.VMEM((16,), jnp.int32),
            pltpu.VMEM((_NCH + 1, _CHUNK), jnp.int32),
            pltpu.VMEM((_CHUNK, _CH), jnp.float32),
            pltpu.VMEM((16,), jnp.int32),
            pltpu.SemaphoreType.DMA,
        ],
        compiler_params=pltpu.CompilerParams(needs_layout_passes=False),
    )
    idx_all, counts, xc = sc_a(nt_pad, x, itemv)

    nt2 = node_type.reshape(_GE, 1, _BE)
    out_base = pl.pallas_call(
        _tc_emb_body,
        grid=(_GE,),
        in_specs=[
            pl.BlockSpec((1, 1, _BE), lambda i: (i, 0, 0)),
            pl.BlockSpec((_NT, _CH), lambda i: (0, 0)),
        ],
        out_specs=pl.BlockSpec((_BE, _CH), lambda i: (i, 0)),
        out_shape=jax.ShapeDtypeStruct((_N, _CH), jnp.float32),
        compiler_params=pltpu.CompilerParams(
            dimension_semantics=("arbitrary",),
        ),
    )(nt2, emb_weight)

    linc = pl.pallas_call(
        _tc_mm_body,
        grid_spec=pltpu.PrefetchScalarGridSpec(
            num_scalar_prefetch=1,
            grid=(_NW, _GM),
            in_specs=[
                pl.BlockSpec((_BM, _CH), _mm_index),
                pl.BlockSpec((_CH, _CH), lambda w, j, c: (0, 0)),
                pl.BlockSpec((1, _CH), lambda w, j, c: (0, 0)),
            ],
            out_specs=pl.BlockSpec((_BM, _CH), _mm_index),
        ),
        out_shape=jax.ShapeDtypeStruct((_N2, _CH), jnp.float32),
        compiler_params=pltpu.CompilerParams(
            dimension_semantics=("arbitrary", "arbitrary"),
        ),
    )(counts, xc, wt, b2)

    out_ref = jax.new_ref(out_base)
    sc_c = pl.kernel(
        _sc_scatter,
        out_type=(),
        mesh=_mesh(),
        scratch_types=[
            pltpu.VMEM((_NCH, _CHUNK), jnp.int32),
            pltpu.VMEM((_CHUNK, _CH), jnp.float32),
            pltpu.VMEM((16,), jnp.int32),
            pltpu.SemaphoreType.DMA,
        ],
        compiler_params=pltpu.CompilerParams(needs_layout_passes=False),
    )
    sc_c(out_ref, linc, idx_all, counts)
    return out_ref[...]


# speedup vs baseline: 2.6998x; 2.6998x over previous
import jax
import jax.numpy as jnp
from jax import lax
from jax.experimental import pallas as pl
from jax.experimental.pallas import tpu as pltpu

_N = 100000
_CH = 128
_NT = 16
_B = 20000
_G = _N // _B

_DN = (((0,), (0,)), ((), ()))


def _body(nt_ref, x_ref, embz_ref, eitem_ref, wt_ref, b_ref, out_ref):
    nt_row = nt_ref[0]
    ohT = (nt_row == lax.broadcasted_iota(jnp.int32, (_NT, _B), 0)
           ).astype(jnp.float32)
    emb_rows = lax.dot_general(ohT, embz_ref[...], _DN,
                               preferred_element_type=jnp.float32)
    maskf = lax.dot_general(ohT, eitem_ref[...], _DN,
                            preferred_element_type=jnp.float32)
    lin = jnp.dot(x_ref[...], wt_ref[...],
                  preferred_element_type=jnp.float32) + b_ref[0, :]
    out_ref[...] = emb_rows + maskf * lin


def kernel(x, node_type, item_id, emb_weight, W, b):
    item32 = jnp.asarray(item_id, jnp.int32)
    sel = (lax.iota(jnp.int32, _NT) == item32).astype(jnp.float32)[:, None]
    emb_z = emb_weight * (1.0 - sel)
    eitem = jnp.broadcast_to(sel, (_NT, _CH))
    nt2 = node_type.reshape(_G, 1, _B)
    wt = W.T
    b2 = b.reshape(1, _CH)
    return pl.pallas_call(
        _body,
        grid=(_G,),
        in_specs=[
            pl.BlockSpec((1, 1, _B), lambda i: (i, 0, 0)),
            pl.BlockSpec((_B, _CH), lambda i: (i, 0)),
            pl.BlockSpec((_NT, _CH), lambda i: (0, 0)),
            pl.BlockSpec((_NT, _CH), lambda i: (0, 0)),
            pl.BlockSpec((_CH, _CH), lambda i: (0, 0)),
            pl.BlockSpec((1, _CH), lambda i: (0, 0)),
        ],
        out_specs=pl.BlockSpec((_B, _CH), lambda i: (i, 0)),
        out_shape=jax.ShapeDtypeStruct((_N, _CH), jnp.float32),
        compiler_params=pltpu.CompilerParams(
            dimension_semantics=("arbitrary",),
        ),
    )(nt2, x, emb_z, eitem, wt, b2)


# speedup vs baseline: 2.9640x; 1.0979x over previous
import jax
import jax.numpy as jnp
from jax import lax
from jax.experimental import pallas as pl
from jax.experimental.pallas import tpu as pltpu

_N = 100000
_CH = 128
_NT = 16
_B = 20000
_G = _N // _B

_DN = (((0,), (0,)), ((), ()))
_DNT = (((1,), (1,)), ((), ()))


def _body(item_ref, nt_ref, x_ref, emb_ref, w_ref, b_ref, out_ref):
    nt_row = nt_ref[0]
    ohT = (nt_row == lax.broadcasted_iota(jnp.int32, (_NT, _B), 0)
           ).astype(jnp.float32)
    selT = (lax.broadcasted_iota(jnp.int32, (_NT, _CH), 0) == item_ref[0]
            ).astype(jnp.float32)
    emb_rows = lax.dot_general(ohT, emb_ref[...] * (1.0 - selT), _DN,
                               preferred_element_type=jnp.float32)
    maskf = lax.dot_general(ohT, selT, _DN,
                            preferred_element_type=jnp.float32)
    lin = lax.dot_general(x_ref[...], w_ref[...], _DNT,
                          preferred_element_type=jnp.float32) + b_ref[0, :]
    out_ref[...] = emb_rows + maskf * lin


def kernel(x, node_type, item_id, emb_weight, W, b):
    item1 = jnp.asarray(item_id, jnp.int32).reshape(1)
    nt2 = node_type.reshape(_G, 1, _B)
    b2 = b.reshape(1, _CH)
    return pl.pallas_call(
        _body,
        grid=(_G,),
        in_specs=[
            pl.BlockSpec(memory_space=pltpu.SMEM),
            pl.BlockSpec((1, 1, _B), lambda i: (i, 0, 0)),
            pl.BlockSpec((_B, _CH), lambda i: (i, 0)),
            pl.BlockSpec((_NT, _CH), lambda i: (0, 0)),
            pl.BlockSpec((_CH, _CH), lambda i: (0, 0)),
            pl.BlockSpec((1, _CH), lambda i: (0, 0)),
        ],
        out_specs=pl.BlockSpec((_B, _CH), lambda i: (i, 0)),
        out_shape=jax.ShapeDtypeStruct((_N, _CH), jnp.float32),
        compiler_params=pltpu.CompilerParams(
            dimension_semantics=("arbitrary",),
        ),
    )(item1, nt2, x, emb_weight, W, b2)
